# baseline (speedup 1.0000x reference)
import jax
import jax.numpy as jnp
from jax.experimental import pallas as pl
from jax.experimental.pallas import tpu as pltpu


def _support_body(x_ref, w1_ref, out_ref):
    out_ref[...] = jnp.dot(x_ref[...], w1_ref[...],
                           preferred_element_type=jnp.float32)


def _pass1_body(adj_ref, s_ref, b1_ref, w2_ref, g_ref):
    h = jnp.dot(adj_ref[...], s_ref[...], preferred_element_type=jnp.float32)
    h = jnp.maximum(h + b1_ref[...], 0.0)
    g_ref[...] = jnp.dot(h, w2_ref[...], preferred_element_type=jnp.float32)


def _pass2_body(adj_ref, g_ref, b2_ref, out_ref):
    v = jnp.dot(adj_ref[...], g_ref[...], preferred_element_type=jnp.float32)
    v = v + b2_ref[...]
    m = jnp.max(v, axis=1, keepdims=True)
    lse = jnp.log(jnp.sum(jnp.exp(v - m), axis=1, keepdims=True)) + m
    out_ref[...] = v - lse


def kernel(x, adj, W1, b1, W2, b2):
    n, nfeat = x.shape
    nhid = W1.shape[1]
    nclass = W2.shape[1]
    b1r = b1.reshape(1, nhid)
    b2r = b2.reshape(1, nclass)

    tm = 400
    grid = (n // tm,)
    params = pltpu.CompilerParams(dimension_semantics=("parallel",))

    support = pl.pallas_call(
        _support_body,
        grid=grid,
        in_specs=[
            pl.BlockSpec((tm, nfeat), lambda i: (i, 0)),
            pl.BlockSpec((nfeat, nhid), lambda i: (0, 0)),
        ],
        out_specs=pl.BlockSpec((tm, nhid), lambda i: (i, 0)),
        out_shape=jax.ShapeDtypeStruct((n, nhid), jnp.float32),
        compiler_params=params,
    )(x, W1)

    g = pl.pallas_call(
        _pass1_body,
        grid=grid,
        in_specs=[
            pl.BlockSpec((tm, n), lambda i: (i, 0)),
            pl.BlockSpec((n, nhid), lambda i: (0, 0)),
            pl.BlockSpec((1, nhid), lambda i: (0, 0)),
            pl.BlockSpec((nhid, nclass), lambda i: (0, 0)),
        ],
        out_specs=pl.BlockSpec((tm, nclass), lambda i: (i, 0)),
        out_shape=jax.ShapeDtypeStruct((n, nclass), jnp.float32),
        compiler_params=params,
    )(adj, support, b1r, W2)

    out = pl.pallas_call(
        _pass2_body,
        grid=grid,
        in_specs=[
            pl.BlockSpec((tm, n), lambda i: (i, 0)),
            pl.BlockSpec((n, nclass), lambda i: (0, 0)),
            pl.BlockSpec((1, nclass), lambda i: (0, 0)),
        ],
        out_specs=pl.BlockSpec((tm, nclass), lambda i: (i, 0)),
        out_shape=jax.ShapeDtypeStruct((n, nclass), jnp.float32),
        compiler_params=params,
    )(adj, g, b2r)

    return out


# speedup vs baseline: 1.0363x; 1.0363x over previous
import jax
import jax.numpy as jnp
from jax.experimental import pallas as pl
from jax.experimental.pallas import tpu as pltpu

_TM = 400


def _body(x_ref, adj_ref, w1_ref, b1_ref, w2_ref, b2_ref, out_ref, s_ref, g_ref):
    i = pl.program_id(0)
    nblk = pl.num_programs(0) // 3

    @pl.when(i < nblk)
    def _phase0():
        s_ref[pl.ds(i * _TM, _TM), :] = jnp.dot(
            x_ref[...], w1_ref[...], preferred_element_type=jnp.float32)

    @pl.when((i >= nblk) & (i < 2 * nblk))
    def _phase1():
        h = jnp.dot(adj_ref[...], s_ref[...], preferred_element_type=jnp.float32)
        h = jnp.maximum(h + b1_ref[...], 0.0)
        g_ref[pl.ds((i - nblk) * _TM, _TM), :] = jnp.dot(
            h, w2_ref[...], preferred_element_type=jnp.float32)

    @pl.when(i >= 2 * nblk)
    def _phase2():
        v = jnp.dot(adj_ref[...], g_ref[...], preferred_element_type=jnp.float32)
        v = v + b2_ref[...]
        m = jnp.max(v, axis=1, keepdims=True)
        lse = jnp.log(jnp.sum(jnp.exp(v - m), axis=1, keepdims=True)) + m
        out_ref[...] = v - lse


def kernel(x, adj, W1, b1, W2, b2):
    n, nfeat = x.shape
    nhid = W1.shape[1]
    nclass = W2.shape[1]
    b1r = b1.reshape(1, nhid)
    b2r = b2.reshape(1, nclass)

    nblk = n // _TM

    def x_map(i):
        return (jnp.minimum(i, nblk - 1), 0)

    def adj_map(i):
        j = jnp.where(i < nblk, 0, jnp.where(i < 2 * nblk, i - nblk, i - 2 * nblk))
        return (j, 0)

    def out_map(i):
        return (jnp.maximum(i - 2 * nblk, 0), 0)

    const = lambda i: (0, 0)

    out = pl.pallas_call(
        _body,
        grid=(3 * nblk,),
        in_specs=[
            pl.BlockSpec((_TM, nfeat), x_map),
            pl.BlockSpec((_TM, n), adj_map),
            pl.BlockSpec((nfeat, nhid), const),
            pl.BlockSpec((1, nhid), const),
            pl.BlockSpec((nhid, nclass), const),
            pl.BlockSpec((1, nclass), const),
        ],
        out_specs=pl.BlockSpec((_TM, nclass), out_map),
        out_shape=jax.ShapeDtypeStruct((n, nclass), jnp.float32),
        scratch_shapes=[
            pltpu.VMEM((n, nhid), jnp.float32),
            pltpu.VMEM((n, nclass), jnp.float32),
        ],
        compiler_params=pltpu.CompilerParams(
            dimension_semantics=("arbitrary",)),
    )(x, adj, W1, b1r, W2, b2r)

    return out


# speedup vs baseline: 1.0482x; 1.0114x over previous
import jax
import jax.numpy as jnp
from jax.experimental import pallas as pl
from jax.experimental.pallas import tpu as pltpu

_TM = 400


def _body(x_ref, adj_ref, w1_ref, b1_ref, w2_ref, b2_ref, out_ref, s_ref, g_ref):
    i = pl.program_id(0)
    nblk = pl.num_programs(0) // 3

    @pl.when(i < nblk)
    def _phase0():
        s_ref[pl.ds(i * _TM, _TM), :] = jnp.dot(
            x_ref[...], w1_ref[...], preferred_element_type=jnp.float32)

    @pl.when((i >= nblk) & (i < 2 * nblk))
    def _phase1():
        h = jnp.dot(adj_ref[...], s_ref[...], preferred_element_type=jnp.float32)
        h = jnp.maximum(h + b1_ref[...], 0.0)
        g_ref[pl.ds((i - nblk) * _TM, _TM), :] = jnp.dot(
            h, w2_ref[...], preferred_element_type=jnp.float32)

    @pl.when(i >= 2 * nblk)
    def _phase2():
        v = jnp.dot(adj_ref[...], g_ref[...], preferred_element_type=jnp.float32)
        v = v + b2_ref[...]
        m = jnp.max(v, axis=1, keepdims=True)
        lse = jnp.log(jnp.sum(jnp.exp(v - m), axis=1, keepdims=True)) + m
        out_ref[...] = v - lse


def kernel(x, adj, W1, b1, W2, b2):
    n, nfeat = x.shape
    nhid = W1.shape[1]
    nclass = W2.shape[1]
    b1r = b1.reshape(1, nhid)
    b2r = b2.reshape(1, nclass)

    nblk = n // _TM

    def x_map(i):
        return (jnp.minimum(i, nblk - 1), 0)

    def adj_map(i):
        j = jnp.where(i < nblk, 0, jnp.where(i < 2 * nblk, i - nblk, i - 2 * nblk))
        return (j, 0)

    def out_map(i):
        return (jnp.maximum(i - 2 * nblk, 0), 0)

    const = lambda i: (0, 0)

    out = pl.pallas_call(
        _body,
        grid=(3 * nblk,),
        in_specs=[
            pl.BlockSpec((_TM, nfeat), x_map),
            pl.BlockSpec((_TM, n), adj_map),
            pl.BlockSpec((nfeat, nhid), const),
            pl.BlockSpec((1, nhid), const),
            pl.BlockSpec((nhid, nclass), const),
            pl.BlockSpec((1, nclass), const),
        ],
        out_specs=pl.BlockSpec((_TM, nclass), out_map),
        out_shape=jax.ShapeDtypeStruct((n, nclass), jnp.float32),
        scratch_shapes=[
            pltpu.VMEM((n, nhid), jnp.float32),
            pltpu.VMEM((n, nclass), jnp.float32),
        ],
        compiler_params=pltpu.CompilerParams(
            dimension_semantics=("arbitrary",),
            vmem_limit_bytes=116 * 1024 * 1024),
    )(x, adj, W1, b1r, W2, b2r)

    return out
